# Initial kernel scaffold; baseline (speedup 1.0000x reference)
import jax
import jax.numpy as jnp
from jax.experimental import pallas as pl


def kernel(input, table):
    raise NotImplementedError("write your pallas kernel here")


# speedup vs baseline: 1.6851x; 1.6851x over previous
import functools

import jax
import jax.numpy as jnp
from jax import lax
from jax.experimental import pallas as pl
from jax.experimental.pallas import tpu as pltpu
from jax.experimental.pallas import tpu_sc as plsc

_NUM_WORKERS = 32
_CHUNK = 128


@functools.partial(jax.jit, static_argnames=("n_chunks", "d_model"))
def _embed(idx3, table, n_chunks, d_model):
    mesh = plsc.VectorSubcoreMesh(core_axis_name="c", subcore_axis_name="s")
    n_total = _NUM_WORKERS * n_chunks * _CHUNK

    @functools.partial(
        pl.kernel,
        mesh=mesh,
        out_type=jax.ShapeDtypeStruct((n_total, d_model), jnp.float32),
        scratch_types=[
            pltpu.VMEM((n_chunks, _CHUNK), jnp.int32),
            pltpu.VMEM((_CHUNK, d_model), jnp.float32),
            pltpu.SemaphoreType.DMA,
        ],
        compiler_params=pltpu.CompilerParams(use_tc_tiling_on_sc=False),
    )
    def k(idx_hbm, tab_hbm, out_hbm, idx_v, rows_v, sem):
        cid = lax.axis_index("c")
        sid = lax.axis_index("s")
        wid = sid * 2 + cid
        pltpu.sync_copy(idx_hbm.at[wid], idx_v)
        base = wid * (n_chunks * _CHUNK)

        def body(c, _):
            pltpu.async_copy(tab_hbm.at[idx_v.at[c]], rows_v, sem).wait()
            pltpu.sync_copy(rows_v, out_hbm.at[pl.ds(base + c * _CHUNK, _CHUNK)])
            return _

        lax.fori_loop(0, n_chunks, body, 0)

    return k(idx3, table)


def kernel(input, table):
    b, s = input.shape
    v, d = table.shape
    n = b * s
    assert n % (_NUM_WORKERS * _CHUNK) == 0
    n_chunks = n // (_NUM_WORKERS * _CHUNK)
    idx3 = input.reshape(_NUM_WORKERS, n_chunks, _CHUNK).astype(jnp.int32)
    out = _embed(idx3, table, n_chunks, d)
    return out.reshape(b, s, d)


# speedup vs baseline: 1.8775x; 1.1142x over previous
import functools

import jax
import jax.numpy as jnp
from jax import lax
from jax.experimental import pallas as pl
from jax.experimental.pallas import tpu as pltpu
from jax.experimental.pallas import tpu_sc as plsc

_NUM_WORKERS = 32
_CHUNK = 128
_NBUF = 4


@functools.partial(jax.jit, static_argnames=("n_chunks", "d_model"))
def _embed(idx3, table, n_chunks, d_model):
    mesh = plsc.VectorSubcoreMesh(core_axis_name="c", subcore_axis_name="s")
    n_total = _NUM_WORKERS * n_chunks * _CHUNK
    assert n_chunks % _NBUF == 0 and n_chunks > _NBUF

    @functools.partial(
        pl.kernel,
        mesh=mesh,
        out_type=jax.ShapeDtypeStruct((n_total, d_model), jnp.float32),
        scratch_types=[
            pltpu.VMEM((n_chunks, _CHUNK), jnp.int32),
            [pltpu.VMEM((_CHUNK, d_model), jnp.float32) for _ in range(_NBUF)],
            [pltpu.SemaphoreType.DMA for _ in range(_NBUF)],
            [pltpu.SemaphoreType.DMA for _ in range(_NBUF)],
        ],
        compiler_params=pltpu.CompilerParams(use_tc_tiling_on_sc=False),
    )
    def k(idx_hbm, tab_hbm, out_hbm, idx_v, rows, g_sems, w_sems):
        cid = lax.axis_index("c")
        sid = lax.axis_index("s")
        wid = sid * 2 + cid
        pltpu.sync_copy(idx_hbm.at[wid], idx_v)
        base = wid * (n_chunks * _CHUNK)

        def gather_desc(c, b):
            return pltpu.make_async_copy(tab_hbm.at[idx_v.at[c]], rows[b], g_sems[b])

        def write_desc(c, b):
            return pltpu.make_async_copy(
                rows[b], out_hbm.at[pl.ds(base + c * _CHUNK, _CHUNK)], w_sems[b]
            )

        for b in range(_NBUF - 1):
            gather_desc(b, b).start()

        def body(g, carry):
            for b in range(_NBUF):
                c = g + b
                bb = (b - 1) % _NBUF
                gather_desc(c, b).wait()
                write_desc(c, b).start()

                @pl.when(c >= 1)
                def _():
                    write_desc(c - 1, bb).wait()

                @pl.when(c + _NBUF - 1 <= n_chunks - 1)
                def _():
                    gather_desc(c + _NBUF - 1, bb).start()

            return carry

        lax.fori_loop(0, n_chunks // _NBUF, lambda g, x: body(g * _NBUF, x), 0)
        write_desc(n_chunks - 1, (n_chunks - 1) % _NBUF).wait()

    return k(idx3, table)


def kernel(input, table):
    b, s = input.shape
    v, d = table.shape
    n = b * s
    assert n % (_NUM_WORKERS * _CHUNK) == 0
    n_chunks = n // (_NUM_WORKERS * _CHUNK)
    idx3 = input.reshape(_NUM_WORKERS, n_chunks, _CHUNK).astype(jnp.int32)
    out = _embed(idx3, table, n_chunks, d)
    return out.reshape(b, s, d)
